# Initial kernel scaffold; baseline (speedup 1.0000x reference)
import jax
import jax.numpy as jnp
from jax.experimental import pallas as pl


def kernel(x, edge_index, W_pre, b_pre, W_gcn, b_gcn, W_ih, W_hh, b_ih, b_hh, W_lin, b_lin):
    raise NotImplementedError("write your pallas kernel here")


# speedup vs baseline: 133.7797x; 133.7797x over previous
import functools

import jax
import jax.numpy as jnp
from jax import lax
from jax.experimental import pallas as pl
from jax.experimental.pallas import tpu as pltpu
from jax.experimental.pallas import tpu_sc as plsc

_NUM_ENVS = 1024
_NUM_AGENTS = 32
_D_IN = 128
_D_GCN = 64
_E_PER = 512

_NW = 32
_EPW = _NUM_ENVS // _NW
_AA = _NUM_AGENTS * _NUM_AGENTS


def _sc_edge_counts(ei_flat):
    mesh = plsc.VectorSubcoreMesh(core_axis_name="c", subcore_axis_name="s")

    @functools.partial(
        pl.kernel,
        mesh=mesh,
        compiler_params=pltpu.CompilerParams(needs_layout_passes=False),
        out_type=jax.ShapeDtypeStruct((_NUM_ENVS * _AA,), jnp.float32),
        scratch_types=[
            pltpu.VMEM((_EPW * 2 * _E_PER,), jnp.int32),
            pltpu.VMEM((_EPW * _AA,), jnp.float32),
        ],
    )
    def k(ei_hbm, out_hbm, ei_v, acc_v):
        wid = lax.axis_index("s") * 2 + lax.axis_index("c")
        base = wid * _EPW
        pltpu.sync_copy(ei_hbm.at[pl.ds(base * 2 * _E_PER, _EPW * 2 * _E_PER)], ei_v)

        zeros = jnp.zeros((16,), jnp.float32)
        ones = jnp.ones((16,), jnp.float32)
        iota = lax.iota(jnp.int32, 16)

        def zero_chunk(c, carry):
            acc_v[pl.ds(c * 16, 16)] = zeros
            return carry

        lax.fori_loop(0, _EPW * _AA // 16, zero_chunk, 0)

        def do_env(e, carry):
            ebase = e * 2 * _E_PER
            abase = e * _AA

            def do_chunk(c, carry2):
                r16 = ei_v[pl.ds(ebase + c * 16, 16)]
                c16 = ei_v[pl.ds(ebase + _E_PER + c * 16, 16)]
                idx = c16 * _NUM_AGENTS + r16 + abase
                plsc.addupdate_scatter(acc_v, [idx], ones)
                return carry2

            lax.fori_loop(0, _E_PER // 16, do_chunk, 0)
            plsc.addupdate_scatter(acc_v, [iota * 33 + abase], ones)
            plsc.addupdate_scatter(acc_v, [(iota + 16) * 33 + abase], ones)
            return carry

        lax.fori_loop(0, _EPW, do_env, 0)
        pltpu.sync_copy(acc_v, out_hbm.at[pl.ds(base * _AA, _EPW * _AA)])

    return k(ei_flat)


_E_B = 16


def _tc_forward(x2, C3, W_xh, b_h, W_ir, W_iz, W_in, c_r, c_z, c_n, b_hn, Wl3, b_l):
    grid = _NUM_ENVS // _E_B
    R = _E_B * _NUM_AGENTS

    def body(x_ref, c_ref, wxh_ref, bh_ref, wir_ref, wiz_ref, win_ref,
             cr_ref, cz_ref, cn_ref, bhn_ref, wl_ref, bl_ref,
             val_ref, hid_ref):
        xb = x_ref[...]
        h = jnp.dot(xb, wxh_ref[...], preferred_element_type=jnp.float32) + bh_ref[...]
        Cb = c_ref[...]
        deg = jnp.sum(Cb, axis=2)
        dis = lax.rsqrt(deg)
        h3 = h.reshape(_E_B, _NUM_AGENTS, _D_GCN) * dis[:, :, None]
        m = lax.dot_general(Cb, h3, (((2,), (1,)), ((0,), (0,))),
                            preferred_element_type=jnp.float32)
        m = m * dis[:, :, None]
        m2 = m.reshape(R, _D_GCN)
        r = jax.nn.sigmoid(jnp.dot(m2, wir_ref[...], preferred_element_type=jnp.float32) + cr_ref[...])
        z = jax.nn.sigmoid(jnp.dot(m2, wiz_ref[...], preferred_element_type=jnp.float32) + cz_ref[...])
        n = jnp.tanh(jnp.dot(m2, win_ref[...], preferred_element_type=jnp.float32)
                     + cn_ref[...] + r * bhn_ref[...])
        hn = (1.0 - z) * n
        hid_ref[...] = hn
        t = jnp.sum(hn.reshape(_E_B, _NUM_AGENTS, _D_GCN) * wl_ref[...], axis=2)
        val_ref[...] = jnp.sum(t, axis=1, keepdims=True) + bl_ref[...]

    full = lambda shape: pl.BlockSpec(shape, lambda i: (0,) * len(shape))
    return pl.pallas_call(
        body,
        grid=(grid,),
        in_specs=[
            pl.BlockSpec((R, _D_IN), lambda i: (i, 0)),
            pl.BlockSpec((_E_B, _NUM_AGENTS, _NUM_AGENTS), lambda i: (i, 0, 0)),
            full((_D_IN, _D_GCN)),
            full((1, _D_GCN)),
            full((_D_GCN, _D_GCN)),
            full((_D_GCN, _D_GCN)),
            full((_D_GCN, _D_GCN)),
            full((1, _D_GCN)),
            full((1, _D_GCN)),
            full((1, _D_GCN)),
            full((1, _D_GCN)),
            full((1, _NUM_AGENTS, _D_GCN)),
            full((1, 1)),
        ],
        out_specs=[
            pl.BlockSpec((_E_B, 1), lambda i: (i, 0)),
            pl.BlockSpec((R, _D_GCN), lambda i: (i, 0)),
        ],
        out_shape=[
            jax.ShapeDtypeStruct((_NUM_ENVS, 1), jnp.float32),
            jax.ShapeDtypeStruct((_NUM_ENVS * _NUM_AGENTS, _D_GCN), jnp.float32),
        ],
    )(x2, C3, W_xh, b_h, W_ir, W_iz, W_in, c_r, c_z, c_n, b_hn, Wl3, b_l)


def kernel(x, edge_index, W_pre, b_pre, W_gcn, b_gcn, W_ih, W_hh, b_ih, b_hh, W_lin, b_lin):
    ei_flat = edge_index.astype(jnp.int32).reshape(_NUM_ENVS * 2 * _E_PER)
    C = _sc_edge_counts(ei_flat)
    C3 = C.reshape(_NUM_ENVS, _NUM_AGENTS, _NUM_AGENTS)

    x2 = x.reshape(_NUM_ENVS * _NUM_AGENTS, _D_IN)
    W_xh = W_pre.T @ W_gcn.T
    b_h = (b_pre @ W_gcn.T)[None]
    W_ir = W_ih[:_D_GCN].T
    W_iz = W_ih[_D_GCN:2 * _D_GCN].T
    W_in = W_ih[2 * _D_GCN:].T
    c_r = (b_ih[:_D_GCN] + b_hh[:_D_GCN] + b_gcn @ W_ih[:_D_GCN].T)[None]
    c_z = (b_ih[_D_GCN:2 * _D_GCN] + b_hh[_D_GCN:2 * _D_GCN]
           + b_gcn @ W_ih[_D_GCN:2 * _D_GCN].T)[None]
    c_n = (b_ih[2 * _D_GCN:] + b_gcn @ W_ih[2 * _D_GCN:].T)[None]
    b_hn = b_hh[2 * _D_GCN:][None]
    Wl3 = W_lin.reshape(1, _NUM_AGENTS, _D_GCN)
    b_l = b_lin.reshape(1, 1)

    value, hid = _tc_forward(x2, C3, W_xh, b_h, W_ir, W_iz, W_in,
                             c_r, c_z, c_n, b_hn, Wl3, b_l)
    next_hidden = hid.reshape(_NUM_ENVS, _NUM_AGENTS, _D_GCN)
    return (value, next_hidden)


# speedup vs baseline: 192.6756x; 1.4402x over previous
import functools

import jax
import jax.numpy as jnp
from jax import lax
from jax.experimental import pallas as pl
from jax.experimental.pallas import tpu as pltpu
from jax.experimental.pallas import tpu_sc as plsc

_NUM_ENVS = 1024
_NUM_AGENTS = 32
_D_IN = 128
_D_GCN = 64
_E_PER = 512

_NW = 32
_EPW = _NUM_ENVS // _NW
_AA = _NUM_AGENTS * _NUM_AGENTS


def _sc_edge_counts(ei_flat):
    mesh = plsc.VectorSubcoreMesh(core_axis_name="c", subcore_axis_name="s")

    @functools.partial(
        pl.kernel,
        mesh=mesh,
        compiler_params=pltpu.CompilerParams(needs_layout_passes=False),
        out_type=jax.ShapeDtypeStruct((_NUM_ENVS * _AA,), jnp.float32),
        scratch_types=[
            pltpu.VMEM((_EPW * 2 * _E_PER,), jnp.int32),
            pltpu.VMEM((_EPW * _AA,), jnp.float32),
        ],
    )
    def k(ei_hbm, out_hbm, ei_v, acc_v):
        wid = lax.axis_index("s") * 2 + lax.axis_index("c")
        base = wid * _EPW
        pltpu.sync_copy(ei_hbm.at[pl.ds(base * 2 * _E_PER, _EPW * 2 * _E_PER)], ei_v)

        zeros = jnp.zeros((16,), jnp.float32)
        ones = jnp.ones((16,), jnp.float32)
        iota = lax.iota(jnp.int32, 16)

        def zero_chunk(c, carry):
            acc_v[pl.ds(c * 16, 16)] = zeros
            return carry

        lax.fori_loop(0, _EPW * _AA // 16, zero_chunk, 0)

        def do_env(e, carry):
            ebase = e * 2 * _E_PER
            abase = e * _AA

            def do_chunk(c, carry2):
                r16 = ei_v[pl.ds(ebase + c * 16, 16)]
                c16 = ei_v[pl.ds(ebase + _E_PER + c * 16, 16)]
                idx = c16 * _NUM_AGENTS + r16 + abase
                plsc.addupdate_scatter(acc_v, [idx], ones)
                return carry2

            lax.fori_loop(0, _E_PER // 16, do_chunk, 0)
            plsc.addupdate_scatter(acc_v, [iota * 33 + abase], ones)
            plsc.addupdate_scatter(acc_v, [(iota + 16) * 33 + abase], ones)
            return carry

        lax.fori_loop(0, _EPW, do_env, 0)
        pltpu.sync_copy(acc_v, out_hbm.at[pl.ds(base * _AA, _EPW * _AA)])

    return k(ei_flat)


_E_B = 128


def _tc_forward(x2, C3, W_xh, b_h, W_ir, W_iz, W_in, c_r, c_z, c_n, b_hn, Wl3, b_l):
    grid = _NUM_ENVS // _E_B
    R = _E_B * _NUM_AGENTS

    def body(x_ref, c_ref, wxh_ref, bh_ref, wir_ref, wiz_ref, win_ref,
             cr_ref, cz_ref, cn_ref, bhn_ref, wl_ref, bl_ref,
             val_ref, hid_ref):
        xb = x_ref[...]
        h = jnp.dot(xb, wxh_ref[...], preferred_element_type=jnp.float32) + bh_ref[...]
        Cb = c_ref[...]
        deg = jnp.sum(Cb, axis=2)
        dis = lax.rsqrt(deg)
        h3 = h.reshape(_E_B, _NUM_AGENTS, _D_GCN) * dis[:, :, None]
        m = lax.dot_general(Cb, h3, (((2,), (1,)), ((0,), (0,))),
                            preferred_element_type=jnp.float32)
        m = m * dis[:, :, None]
        m2 = m.reshape(R, _D_GCN)
        r = jax.nn.sigmoid(jnp.dot(m2, wir_ref[...], preferred_element_type=jnp.float32) + cr_ref[...])
        z = jax.nn.sigmoid(jnp.dot(m2, wiz_ref[...], preferred_element_type=jnp.float32) + cz_ref[...])
        n = jnp.tanh(jnp.dot(m2, win_ref[...], preferred_element_type=jnp.float32)
                     + cn_ref[...] + r * bhn_ref[...])
        hn = (1.0 - z) * n
        hid_ref[...] = hn
        t = jnp.sum(hn.reshape(_E_B, _NUM_AGENTS, _D_GCN) * wl_ref[...], axis=2)
        val_ref[...] = jnp.sum(t, axis=1, keepdims=True) + bl_ref[...]

    full = lambda shape: pl.BlockSpec(shape, lambda i: (0,) * len(shape))
    return pl.pallas_call(
        body,
        grid=(grid,),
        in_specs=[
            pl.BlockSpec((R, _D_IN), lambda i: (i, 0)),
            pl.BlockSpec((_E_B, _NUM_AGENTS, _NUM_AGENTS), lambda i: (i, 0, 0)),
            full((_D_IN, _D_GCN)),
            full((1, _D_GCN)),
            full((_D_GCN, _D_GCN)),
            full((_D_GCN, _D_GCN)),
            full((_D_GCN, _D_GCN)),
            full((1, _D_GCN)),
            full((1, _D_GCN)),
            full((1, _D_GCN)),
            full((1, _D_GCN)),
            full((1, _NUM_AGENTS, _D_GCN)),
            full((1, 1)),
        ],
        out_specs=[
            pl.BlockSpec((_E_B, 1), lambda i: (i, 0)),
            pl.BlockSpec((R, _D_GCN), lambda i: (i, 0)),
        ],
        out_shape=[
            jax.ShapeDtypeStruct((_NUM_ENVS, 1), jnp.float32),
            jax.ShapeDtypeStruct((_NUM_ENVS * _NUM_AGENTS, _D_GCN), jnp.float32),
        ],
    )(x2, C3, W_xh, b_h, W_ir, W_iz, W_in, c_r, c_z, c_n, b_hn, Wl3, b_l)


def kernel(x, edge_index, W_pre, b_pre, W_gcn, b_gcn, W_ih, W_hh, b_ih, b_hh, W_lin, b_lin):
    ei_flat = edge_index.astype(jnp.int32).reshape(_NUM_ENVS * 2 * _E_PER)
    C = _sc_edge_counts(ei_flat)
    C3 = C.reshape(_NUM_ENVS, _NUM_AGENTS, _NUM_AGENTS)

    x2 = x.reshape(_NUM_ENVS * _NUM_AGENTS, _D_IN)
    W_xh = W_pre.T @ W_gcn.T
    b_h = (b_pre @ W_gcn.T)[None]
    W_ir = W_ih[:_D_GCN].T
    W_iz = W_ih[_D_GCN:2 * _D_GCN].T
    W_in = W_ih[2 * _D_GCN:].T
    c_r = (b_ih[:_D_GCN] + b_hh[:_D_GCN] + b_gcn @ W_ih[:_D_GCN].T)[None]
    c_z = (b_ih[_D_GCN:2 * _D_GCN] + b_hh[_D_GCN:2 * _D_GCN]
           + b_gcn @ W_ih[_D_GCN:2 * _D_GCN].T)[None]
    c_n = (b_ih[2 * _D_GCN:] + b_gcn @ W_ih[2 * _D_GCN:].T)[None]
    b_hn = b_hh[2 * _D_GCN:][None]
    Wl3 = W_lin.reshape(1, _NUM_AGENTS, _D_GCN)
    b_l = b_lin.reshape(1, 1)

    value, hid = _tc_forward(x2, C3, W_xh, b_h, W_ir, W_iz, W_in,
                             c_r, c_z, c_n, b_hn, Wl3, b_l)
    next_hidden = hid.reshape(_NUM_ENVS, _NUM_AGENTS, _D_GCN)
    return (value, next_hidden)
